# baseline (speedup 1.0000x reference)
import functools

import jax
import jax.numpy as jnp
from jax import lax
from jax.experimental import pallas as pl
from jax.experimental.pallas import tpu as pltpu
from jax.experimental.pallas import tpu_sc as plsc

N = 10000
E = 320000
D = 128
EPS = 1e-5

NC = 2
NS = 16
NW = NC * NS
CHUNK = 100
E_PER_TILE = E // NW
NCHUNK = -(-E_PER_TILE // CHUNK)
EDGE_PAD = NCHUNK * CHUNK - E_PER_TILE
ROWS_PER_TILE = 640
N_PAD = 10240

BLK = 512
GRID = N_PAD // BLK


def _sc_deg_body(dst_hbm, ones_hbm, zcol_hbm, deg_out, dst_v, ones_v, deg_sh,
                 ssem):
    c = lax.axis_index("c")
    s = lax.axis_index("s")
    wid = c * NS + s
    pltpu.sync_copy(dst_hbm.at[wid], dst_v)
    pltpu.sync_copy(ones_hbm, ones_v)
    pltpu.sync_copy(zcol_hbm, deg_sh.at[pl.ds(s * ROWS_PER_TILE, ROWS_PER_TILE)])
    plsc.subcore_barrier()

    pltpu.async_copy(ones_v, deg_sh.at[dst_v.at[0]], ssem.at[0], add=True)

    def step(j, carry):
        @pl.when(j + 1 < NCHUNK)
        def _():
            pltpu.async_copy(ones_v, deg_sh.at[dst_v.at[j + 1]],
                             ssem.at[_even(j + 1)], add=True)

        pltpu.make_async_copy(ones_v, deg_sh.at[dst_v.at[j]],
                              ssem.at[_even(j)]).wait()
        return carry

    lax.fori_loop(0, NCHUNK, step, 0)
    plsc.subcore_barrier()
    sl = pl.ds(s * ROWS_PER_TILE, ROWS_PER_TILE)
    pltpu.sync_copy(deg_sh.at[sl], deg_out.at[c, sl])


def _even(j):
    return lax.rem(j, 2)


def _mod3(j):
    return lax.rem(j, 3)


def _sc_deg(dst_r, ones, zcol):
    mesh = plsc.VectorSubcoreMesh(
        core_axis_name="c", subcore_axis_name="s", num_cores=NC, num_subcores=NS
    )
    f = pl.kernel(
        _sc_deg_body,
        out_type=jax.ShapeDtypeStruct((NC, N_PAD), jnp.float32),
        mesh=mesh,
        scratch_types=[
            pltpu.VMEM((NCHUNK, CHUNK), jnp.int32),
            pltpu.VMEM((CHUNK,), jnp.float32),
            pltpu.VMEM_SHARED((N_PAD,), jnp.float32),
            pltpu.SemaphoreType.DMA((2,)),
        ],
    )
    return f(dst_r, ones, zcol)


def _mod4(j):
    return lax.rem(j, 4)


def _sc_scat_body(y_hbm, src_hbm, dst_hbm, zeros_hbm, out_hbm,
                  srcbuf, dstbuf, rows_v, acc_sh, srcsem, dstsem, gsem, ssem):
    c = lax.axis_index("c")
    s = lax.axis_index("s")
    wid = c * NS + s

    def fetch(j, islot):
        pltpu.async_copy(src_hbm.at[wid, j], srcbuf.at[islot], srcsem.at[islot])
        pltpu.async_copy(dst_hbm.at[wid, j], dstbuf.at[islot], dstsem.at[islot])

    def gather(j, islot, rslot):
        pltpu.make_async_copy(src_hbm.at[wid, j], srcbuf.at[islot],
                              srcsem.at[islot]).wait()
        pltpu.async_copy(y_hbm.at[srcbuf.at[islot]], rows_v.at[rslot],
                         gsem.at[rslot])

    pltpu.async_copy(zeros_hbm,
                     acc_sh.at[pl.ds(s * ROWS_PER_TILE, ROWS_PER_TILE)],
                     ssem.at[2])
    fetch(0, 0)
    fetch(1, 1)
    gather(0, 0, 0)
    pltpu.make_async_copy(zeros_hbm,
                          acc_sh.at[pl.ds(s * ROWS_PER_TILE, ROWS_PER_TILE)],
                          ssem.at[2]).wait()
    plsc.subcore_barrier()

    def step(j, carry):
        @pl.when(j >= 2)
        def _():
            pslot = _mod3(j - 2)
            pltpu.make_async_copy(rows_v.at[pslot],
                                  acc_sh.at[dstbuf.at[_mod4(j - 2)]],
                                  ssem.at[pslot]).wait()

        @pl.when(j + 2 < NCHUNK)
        def _():
            fetch(j + 2, _mod4(j + 2))

        @pl.when(j + 1 < NCHUNK)
        def _():
            gather(j + 1, _mod4(j + 1), _mod3(j + 1))

        rslot = _mod3(j)
        islot = _mod4(j)
        pltpu.make_async_copy(y_hbm.at[srcbuf.at[islot]], rows_v.at[rslot],
                              gsem.at[rslot]).wait()
        pltpu.make_async_copy(dst_hbm.at[wid, j], dstbuf.at[islot],
                              dstsem.at[islot]).wait()
        pltpu.async_copy(rows_v.at[rslot], acc_sh.at[dstbuf.at[islot]],
                         ssem.at[rslot], add=True)
        return carry

    lax.fori_loop(0, NCHUNK, step, 0)
    for j in (NCHUNK - 2, NCHUNK - 1):
        pltpu.make_async_copy(rows_v.at[j % 3],
                              acc_sh.at[dstbuf.at[j % 4]],
                              ssem.at[j % 3]).wait()
    plsc.subcore_barrier()
    sl = pl.ds(s * ROWS_PER_TILE, ROWS_PER_TILE)
    pltpu.sync_copy(acc_sh.at[sl], out_hbm.at[c, sl])


def _sc_scatter(y, src_r, dst_r, zeros):
    mesh = plsc.VectorSubcoreMesh(
        core_axis_name="c", subcore_axis_name="s", num_cores=NC, num_subcores=NS
    )
    f = pl.kernel(
        _sc_scat_body,
        out_type=jax.ShapeDtypeStruct((NC, N_PAD, D), jnp.float32),
        mesh=mesh,
        scratch_types=[
            pltpu.VMEM((4, CHUNK), jnp.int32),
            pltpu.VMEM((4, CHUNK), jnp.int32),
            pltpu.VMEM((3, CHUNK, D), jnp.float32),
            pltpu.VMEM_SHARED((N_PAD, D), jnp.float32),
            pltpu.SemaphoreType.DMA((4,)),
            pltpu.SemaphoreType.DMA((4,)),
            pltpu.SemaphoreType.DMA((3,)),
            pltpu.SemaphoreType.DMA((3,)),
        ],
    )
    return f(y, src_r, dst_r, zeros)


def _tc_mm_body(x_ref, w_ref, xw_ref):
    xw_ref[...] = jnp.dot(x_ref[...], w_ref[...],
                          preferred_element_type=jnp.float32)


def _tc_mm(x, W):
    return pl.pallas_call(
        _tc_mm_body,
        grid=(GRID,),
        in_specs=[
            pl.BlockSpec((BLK, D), lambda i: (i, 0)),
            pl.BlockSpec((D, D), lambda i: (0, 0)),
        ],
        out_specs=pl.BlockSpec((BLK, D), lambda i: (i, 0)),
        out_shape=jax.ShapeDtypeStruct((N, D), jnp.float32),
    )(x, W)


def _tc_scale_body(xw_ref, deg_ref, y_ref):
    deg = deg_ref[0, :] + deg_ref[1, :] + 1.0
    dinv = lax.rsqrt(deg)
    y_ref[...] = xw_ref[...] * dinv[:, None]


def _tc_scale(xw, deg_part):
    return pl.pallas_call(
        _tc_scale_body,
        grid=(GRID,),
        in_specs=[
            pl.BlockSpec((BLK, D), lambda i: (i, 0)),
            pl.BlockSpec((NC, BLK), lambda i: (0, i)),
        ],
        out_specs=pl.BlockSpec((BLK, D), lambda i: (i, 0)),
        out_shape=jax.ShapeDtypeStruct((N, D), jnp.float32),
    )(xw, deg_part)


def _p1(i):
    return jnp.minimum(i, GRID - 1)


def _p2(i):
    return jnp.maximum(i - GRID, 0)
def _tc_fin_body(acc_ref, y_ref, deg_ref, b_ref, x_ref, w_ref, bias_ref,
                 ms_ref, o_ref, h_sc, s1, s2):
    i = pl.program_id(0)
    ip = lax.rem(i, GRID)

    @pl.when(i == 0)
    def _():
        s1[...] = jnp.zeros_like(s1)
        s2[...] = jnp.zeros_like(s2)

    @pl.when(i < GRID)
    def _():
        deg = deg_ref[0, :] + deg_ref[1, :] + 1.0
        dinv = lax.rsqrt(deg)
        h = dinv[:, None] * (acc_ref[0] + acc_ref[1] + y_ref[...]) + b_ref[...]
        row = ip * BLK + lax.broadcasted_iota(jnp.int32, (BLK, D), 0)
        hm = jnp.where(row < N, h, 0.0)
        h_sc[ip] = h
        s1[...] += jnp.sum(hm, axis=0, keepdims=True)
        s2[...] += jnp.sum(hm * hm, axis=0, keepdims=True)

    @pl.when(i >= GRID)
    def _():
        m = s1[0, :] * (1.0 / N)
        q = s2[0, :] * (1.0 / N)
        ms = ms_ref[0, :]
        mm = m * ms
        var = q - 2.0 * mm * m + mm * mm
        scale = w_ref[0, :] * lax.rsqrt(var + EPS)
        h = h_sc[ip]
        o = (h - mm[None, :]) * scale[None, :] + bias_ref[0, :][None, :]
        o_ref[...] = jnp.maximum(o, 0.0) + x_ref[...]


def _tc_finish(acc_part, y, deg_part, b2, x, gw2, gb2, gms2):
    return pl.pallas_call(
        _tc_fin_body,
        grid=(2 * GRID,),
        in_specs=[
            pl.BlockSpec((NC, BLK, D), lambda i: (0, _p1(i), 0)),
            pl.BlockSpec((BLK, D), lambda i: (_p1(i), 0)),
            pl.BlockSpec((NC, BLK), lambda i: (0, _p1(i))),
            pl.BlockSpec((1, D), lambda i: (0, 0)),
            pl.BlockSpec((BLK, D), lambda i: (_p2(i), 0)),
            pl.BlockSpec((1, D), lambda i: (0, 0)),
            pl.BlockSpec((1, D), lambda i: (0, 0)),
            pl.BlockSpec((1, D), lambda i: (0, 0)),
        ],
        out_specs=pl.BlockSpec((BLK, D), lambda i: (_p2(i), 0)),
        out_shape=jax.ShapeDtypeStruct((N, D), jnp.float32),
        scratch_shapes=[
            pltpu.VMEM((GRID, BLK, D), jnp.float32),
            pltpu.VMEM((1, D), jnp.float32),
            pltpu.VMEM((1, D), jnp.float32),
        ],
    )(acc_part, y, deg_part, b2, x, gw2, gb2, gms2)


def kernel(x, edge_index, W, b, gn_weight, gn_bias, gn_mean_scale):
    er = edge_index.reshape(2, NW, NCHUNK, CHUNK)
    src_r, dst_r = er[0], er[1]

    ones = jnp.ones((CHUNK,), jnp.float32)
    zcol = jnp.zeros((ROWS_PER_TILE,), jnp.float32)
    zeros = jnp.zeros((ROWS_PER_TILE, D), jnp.float32)

    xw = _tc_mm(x, W)
    deg_part = _sc_deg(dst_r, ones, zcol)
    y = _tc_scale(xw, deg_part)
    acc_part = _sc_scatter(y, src_r, dst_r, zeros)
    return _tc_finish(acc_part, y, deg_part, b.reshape(1, D), x,
                      gn_weight.reshape(1, D), gn_bias.reshape(1, D),
                      gn_mean_scale.reshape(1, D))


# speedup vs baseline: 1.1165x; 1.1165x over previous
import functools

import jax
import jax.numpy as jnp
from jax import lax
from jax.experimental import pallas as pl
from jax.experimental.pallas import tpu as pltpu
from jax.experimental.pallas import tpu_sc as plsc

N = 10000
E = 320000
D = 128
EPS = 1e-5

NC = 2
NS = 16
NW = NC * NS
CHUNK = 100
E_PER_TILE = E // NW
NCHUNK = -(-E_PER_TILE // CHUNK)
EDGE_PAD = NCHUNK * CHUNK - E_PER_TILE
ROWS_PER_TILE = 640
N_PAD = 10240

BLK = 512
GRID = N_PAD // BLK


def _sc_deg_body(er_hbm, ones_hbm, zcol_hbm, deg_out, dst_v, ones_v, deg_sh,
                 ssem):
    c = lax.axis_index("c")
    s = lax.axis_index("s")
    wid = c * NS + s
    pltpu.sync_copy(er_hbm.at[1, wid], dst_v)
    pltpu.sync_copy(ones_hbm, ones_v)
    pltpu.sync_copy(zcol_hbm, deg_sh.at[pl.ds(s * ROWS_PER_TILE, ROWS_PER_TILE)])
    plsc.subcore_barrier()

    pltpu.async_copy(ones_v, deg_sh.at[dst_v.at[0]], ssem.at[0], add=True)

    def step(j, carry):
        @pl.when(j + 1 < NCHUNK)
        def _():
            pltpu.async_copy(ones_v, deg_sh.at[dst_v.at[j + 1]],
                             ssem.at[_even(j + 1)], add=True)

        pltpu.make_async_copy(ones_v, deg_sh.at[dst_v.at[j]],
                              ssem.at[_even(j)]).wait()
        return carry

    lax.fori_loop(0, NCHUNK, step, 0)
    plsc.subcore_barrier()
    sl = pl.ds(s * ROWS_PER_TILE, ROWS_PER_TILE)
    pltpu.sync_copy(deg_sh.at[sl], deg_out.at[c, sl])


def _even(j):
    return lax.rem(j, 2)


def _mod3(j):
    return lax.rem(j, 3)


def _sc_deg(er, ones, zcol):
    mesh = plsc.VectorSubcoreMesh(
        core_axis_name="c", subcore_axis_name="s", num_cores=NC, num_subcores=NS
    )
    f = pl.kernel(
        _sc_deg_body,
        out_type=jax.ShapeDtypeStruct((NC, N_PAD), jnp.float32),
        mesh=mesh,
        scratch_types=[
            pltpu.VMEM((NCHUNK, CHUNK), jnp.int32),
            pltpu.VMEM((CHUNK,), jnp.float32),
            pltpu.VMEM_SHARED((N_PAD,), jnp.float32),
            pltpu.SemaphoreType.DMA((2,)),
        ],
    )
    return f(er, ones, zcol)


def _mod4(j):
    return lax.rem(j, 4)


def _sc_scat_body(y_hbm, er_hbm, zeros_hbm, out_hbm,
                  srcbuf, dstbuf, rows_v, acc_sh, srcsem, dstsem, gsem, ssem):
    c = lax.axis_index("c")
    s = lax.axis_index("s")
    wid = c * NS + s

    def fetch(j, islot):
        pltpu.async_copy(er_hbm.at[0, wid, j], srcbuf.at[islot],
                         srcsem.at[islot])
        pltpu.async_copy(er_hbm.at[1, wid, j], dstbuf.at[islot],
                         dstsem.at[islot])

    def gather(j, islot, rslot):
        pltpu.make_async_copy(er_hbm.at[0, wid, j], srcbuf.at[islot],
                              srcsem.at[islot]).wait()
        pltpu.async_copy(y_hbm.at[srcbuf.at[islot]], rows_v.at[rslot],
                         gsem.at[rslot])

    pltpu.async_copy(zeros_hbm,
                     acc_sh.at[pl.ds(s * ROWS_PER_TILE, ROWS_PER_TILE)],
                     ssem.at[2])
    fetch(0, 0)
    fetch(1, 1)
    gather(0, 0, 0)
    pltpu.make_async_copy(zeros_hbm,
                          acc_sh.at[pl.ds(s * ROWS_PER_TILE, ROWS_PER_TILE)],
                          ssem.at[2]).wait()
    plsc.subcore_barrier()

    def step(j, carry):
        @pl.when(j >= 2)
        def _():
            pslot = _mod3(j - 2)
            pltpu.make_async_copy(rows_v.at[pslot],
                                  acc_sh.at[dstbuf.at[_mod4(j - 2)]],
                                  ssem.at[pslot]).wait()

        @pl.when(j + 2 < NCHUNK)
        def _():
            fetch(j + 2, _mod4(j + 2))

        @pl.when(j + 1 < NCHUNK)
        def _():
            gather(j + 1, _mod4(j + 1), _mod3(j + 1))

        rslot = _mod3(j)
        islot = _mod4(j)
        pltpu.make_async_copy(y_hbm.at[srcbuf.at[islot]], rows_v.at[rslot],
                              gsem.at[rslot]).wait()
        pltpu.make_async_copy(er_hbm.at[1, wid, j], dstbuf.at[islot],
                              dstsem.at[islot]).wait()
        pltpu.async_copy(rows_v.at[rslot], acc_sh.at[dstbuf.at[islot]],
                         ssem.at[rslot], add=True)
        return carry

    lax.fori_loop(0, NCHUNK, step, 0)
    for j in (NCHUNK - 2, NCHUNK - 1):
        pltpu.make_async_copy(rows_v.at[j % 3],
                              acc_sh.at[dstbuf.at[j % 4]],
                              ssem.at[j % 3]).wait()
    plsc.subcore_barrier()
    sl = pl.ds(s * ROWS_PER_TILE, ROWS_PER_TILE)
    pltpu.sync_copy(acc_sh.at[sl], out_hbm.at[c, sl])


def _sc_scatter(y, er, zeros):
    mesh = plsc.VectorSubcoreMesh(
        core_axis_name="c", subcore_axis_name="s", num_cores=NC, num_subcores=NS
    )
    f = pl.kernel(
        _sc_scat_body,
        out_type=jax.ShapeDtypeStruct((NC, N_PAD, D), jnp.float32),
        mesh=mesh,
        scratch_types=[
            pltpu.VMEM((4, CHUNK), jnp.int32),
            pltpu.VMEM((4, CHUNK), jnp.int32),
            pltpu.VMEM((3, CHUNK, D), jnp.float32),
            pltpu.VMEM_SHARED((N_PAD, D), jnp.float32),
            pltpu.SemaphoreType.DMA((4,)),
            pltpu.SemaphoreType.DMA((4,)),
            pltpu.SemaphoreType.DMA((3,)),
            pltpu.SemaphoreType.DMA((3,)),
        ],
    )
    return f(y, er, zeros)


def _tc_prep_body(x_ref, w_ref, deg_ref, y_ref):
    xw = jnp.dot(x_ref[...], w_ref[...], preferred_element_type=jnp.float32)
    deg = deg_ref[0, :] + deg_ref[1, :] + 1.0
    dinv = lax.rsqrt(deg)
    y_ref[...] = xw * dinv[:, None]


def _tc_prep(x, W, deg_part):
    return pl.pallas_call(
        _tc_prep_body,
        grid=(GRID,),
        in_specs=[
            pl.BlockSpec((BLK, D), lambda i: (i, 0)),
            pl.BlockSpec((D, D), lambda i: (0, 0)),
            pl.BlockSpec((NC, BLK), lambda i: (0, i)),
        ],
        out_specs=pl.BlockSpec((BLK, D), lambda i: (i, 0)),
        out_shape=jax.ShapeDtypeStruct((N, D), jnp.float32),
    )(x, W, deg_part)


def _p1(i):
    return jnp.minimum(i, GRID - 1)


def _p2(i):
    return jnp.maximum(i - GRID, 0)
def _tc_fin_body(acc_ref, y_ref, deg_ref, b_ref, x_ref, w_ref, bias_ref,
                 ms_ref, o_ref, h_sc, s1, s2):
    i = pl.program_id(0)
    ip = lax.rem(i, GRID)

    @pl.when(i == 0)
    def _():
        s1[...] = jnp.zeros_like(s1)
        s2[...] = jnp.zeros_like(s2)

    @pl.when(i < GRID)
    def _():
        deg = deg_ref[0, :] + deg_ref[1, :] + 1.0
        dinv = lax.rsqrt(deg)
        h = dinv[:, None] * (acc_ref[0] + acc_ref[1] + y_ref[...]) + b_ref[...]
        row = ip * BLK + lax.broadcasted_iota(jnp.int32, (BLK, D), 0)
        hm = jnp.where(row < N, h, 0.0)
        h_sc[ip] = h
        s1[...] += jnp.sum(hm, axis=0, keepdims=True)
        s2[...] += jnp.sum(hm * hm, axis=0, keepdims=True)

    @pl.when(i >= GRID)
    def _():
        m = s1[0, :] * (1.0 / N)
        q = s2[0, :] * (1.0 / N)
        ms = ms_ref[0, :]
        mm = m * ms
        var = q - 2.0 * mm * m + mm * mm
        scale = w_ref[0, :] * lax.rsqrt(var + EPS)
        h = h_sc[ip]
        o = (h - mm[None, :]) * scale[None, :] + bias_ref[0, :][None, :]
        o_ref[...] = jnp.maximum(o, 0.0) + x_ref[...]


def _tc_finish(acc_part, y, deg_part, b2, x, gw2, gb2, gms2):
    return pl.pallas_call(
        _tc_fin_body,
        grid=(2 * GRID,),
        in_specs=[
            pl.BlockSpec((NC, BLK, D), lambda i: (0, _p1(i), 0)),
            pl.BlockSpec((BLK, D), lambda i: (_p1(i), 0)),
            pl.BlockSpec((NC, BLK), lambda i: (0, _p1(i))),
            pl.BlockSpec((1, D), lambda i: (0, 0)),
            pl.BlockSpec((BLK, D), lambda i: (_p2(i), 0)),
            pl.BlockSpec((1, D), lambda i: (0, 0)),
            pl.BlockSpec((1, D), lambda i: (0, 0)),
            pl.BlockSpec((1, D), lambda i: (0, 0)),
        ],
        out_specs=pl.BlockSpec((BLK, D), lambda i: (_p2(i), 0)),
        out_shape=jax.ShapeDtypeStruct((N, D), jnp.float32),
        scratch_shapes=[
            pltpu.VMEM((GRID, BLK, D), jnp.float32),
            pltpu.VMEM((1, D), jnp.float32),
            pltpu.VMEM((1, D), jnp.float32),
        ],
    )(acc_part, y, deg_part, b2, x, gw2, gb2, gms2)


def kernel(x, edge_index, W, b, gn_weight, gn_bias, gn_mean_scale):
    er = edge_index.reshape(2, NW, NCHUNK, CHUNK)

    ones = jnp.ones((CHUNK,), jnp.float32)
    zcol = jnp.zeros((ROWS_PER_TILE,), jnp.float32)
    zeros = jnp.zeros((ROWS_PER_TILE, D), jnp.float32)

    deg_part = _sc_deg(er, ones, zcol)
    y = _tc_prep(x, W, deg_part)
    acc_part = _sc_scatter(y, er, zeros)
    return _tc_finish(acc_part, y, deg_part, b.reshape(1, D), x,
                      gn_weight.reshape(1, D), gn_bias.reshape(1, D),
                      gn_mean_scale.reshape(1, D))


# speedup vs baseline: 1.2207x; 1.0934x over previous
import functools

import jax
import jax.numpy as jnp
from jax import lax
from jax.experimental import pallas as pl
from jax.experimental.pallas import tpu as pltpu
from jax.experimental.pallas import tpu_sc as plsc

N = 10000
E = 320000
D = 128
EPS = 1e-5

NC = 2
NS = 16
NW = NC * NS
CHUNK = 100
E_PER_TILE = E // NW
NCHUNK = -(-E_PER_TILE // CHUNK)
EDGE_PAD = NCHUNK * CHUNK - E_PER_TILE
ROWS_PER_TILE = 640
N_PAD = 10240

BLK = 1024
GRID = N_PAD // BLK


def _sc_deg_body(er_hbm, ones_hbm, zcol_hbm, deg_out, dst_v, ones_v, deg_sh,
                 ssem):
    c = lax.axis_index("c")
    s = lax.axis_index("s")
    wid = c * NS + s
    pltpu.sync_copy(er_hbm.at[1, wid], dst_v)
    pltpu.sync_copy(ones_hbm, ones_v)
    pltpu.sync_copy(zcol_hbm, deg_sh.at[pl.ds(s * ROWS_PER_TILE, ROWS_PER_TILE)])
    plsc.subcore_barrier()

    pltpu.async_copy(ones_v, deg_sh.at[dst_v.at[0]], ssem.at[0], add=True)

    def step(j, carry):
        @pl.when(j + 1 < NCHUNK)
        def _():
            pltpu.async_copy(ones_v, deg_sh.at[dst_v.at[j + 1]],
                             ssem.at[_even(j + 1)], add=True)

        pltpu.make_async_copy(ones_v, deg_sh.at[dst_v.at[j]],
                              ssem.at[_even(j)]).wait()
        return carry

    lax.fori_loop(0, NCHUNK, step, 0)
    plsc.subcore_barrier()
    sl = pl.ds(s * ROWS_PER_TILE, ROWS_PER_TILE)
    pltpu.sync_copy(deg_sh.at[sl], deg_out.at[c, sl])


def _even(j):
    return lax.rem(j, 2)


def _mod3(j):
    return lax.rem(j, 3)


def _sc_deg(er, ones, zcol):
    mesh = plsc.VectorSubcoreMesh(
        core_axis_name="c", subcore_axis_name="s", num_cores=NC, num_subcores=NS
    )
    f = pl.kernel(
        _sc_deg_body,
        out_type=jax.ShapeDtypeStruct((NC, N_PAD), jnp.float32),
        mesh=mesh,
        scratch_types=[
            pltpu.VMEM((NCHUNK, CHUNK), jnp.int32),
            pltpu.VMEM((CHUNK,), jnp.float32),
            pltpu.VMEM_SHARED((N_PAD,), jnp.float32),
            pltpu.SemaphoreType.DMA((2,)),
        ],
    )
    return f(er, ones, zcol)


def _mod4(j):
    return lax.rem(j, 4)


def _sc_scat_body(y_hbm, er_hbm, zeros_hbm, out_hbm,
                  srcbuf, dstbuf, rows_v, acc_sh, srcsem, dstsem, gsem, ssem):
    c = lax.axis_index("c")
    s = lax.axis_index("s")
    wid = c * NS + s

    def fetch(j, islot):
        pltpu.async_copy(er_hbm.at[0, wid, j], srcbuf.at[islot],
                         srcsem.at[islot])
        pltpu.async_copy(er_hbm.at[1, wid, j], dstbuf.at[islot],
                         dstsem.at[islot])

    def gather(j, islot, rslot):
        pltpu.make_async_copy(er_hbm.at[0, wid, j], srcbuf.at[islot],
                              srcsem.at[islot]).wait()
        pltpu.async_copy(y_hbm.at[srcbuf.at[islot]], rows_v.at[rslot],
                         gsem.at[rslot])

    pltpu.async_copy(zeros_hbm,
                     acc_sh.at[pl.ds(s * ROWS_PER_TILE, ROWS_PER_TILE)],
                     ssem.at[2])
    fetch(0, 0)
    fetch(1, 1)
    gather(0, 0, 0)
    pltpu.make_async_copy(zeros_hbm,
                          acc_sh.at[pl.ds(s * ROWS_PER_TILE, ROWS_PER_TILE)],
                          ssem.at[2]).wait()
    plsc.subcore_barrier()

    def step(j, carry):
        @pl.when(j >= 2)
        def _():
            pslot = _mod3(j - 2)
            pltpu.make_async_copy(rows_v.at[pslot],
                                  acc_sh.at[dstbuf.at[_mod4(j - 2)]],
                                  ssem.at[pslot]).wait()

        @pl.when(j + 2 < NCHUNK)
        def _():
            fetch(j + 2, _mod4(j + 2))

        @pl.when(j + 1 < NCHUNK)
        def _():
            gather(j + 1, _mod4(j + 1), _mod3(j + 1))

        rslot = _mod3(j)
        islot = _mod4(j)
        pltpu.make_async_copy(y_hbm.at[srcbuf.at[islot]], rows_v.at[rslot],
                              gsem.at[rslot]).wait()
        pltpu.make_async_copy(er_hbm.at[1, wid, j], dstbuf.at[islot],
                              dstsem.at[islot]).wait()
        pltpu.async_copy(rows_v.at[rslot], acc_sh.at[dstbuf.at[islot]],
                         ssem.at[rslot], add=True)
        return carry

    lax.fori_loop(0, NCHUNK, step, 0)
    for j in (NCHUNK - 2, NCHUNK - 1):
        pltpu.make_async_copy(rows_v.at[j % 3],
                              acc_sh.at[dstbuf.at[j % 4]],
                              ssem.at[j % 3]).wait()
    plsc.subcore_barrier()
    sl = pl.ds(s * ROWS_PER_TILE, ROWS_PER_TILE)
    pltpu.sync_copy(acc_sh.at[sl], out_hbm.at[c, sl])


def _sc_scatter(y, er, zeros):
    mesh = plsc.VectorSubcoreMesh(
        core_axis_name="c", subcore_axis_name="s", num_cores=NC, num_subcores=NS
    )
    f = pl.kernel(
        _sc_scat_body,
        out_type=jax.ShapeDtypeStruct((NC, N_PAD, D), jnp.float32),
        mesh=mesh,
        scratch_types=[
            pltpu.VMEM((4, CHUNK), jnp.int32),
            pltpu.VMEM((4, CHUNK), jnp.int32),
            pltpu.VMEM((3, CHUNK, D), jnp.float32),
            pltpu.VMEM_SHARED((N_PAD, D), jnp.float32),
            pltpu.SemaphoreType.DMA((4,)),
            pltpu.SemaphoreType.DMA((4,)),
            pltpu.SemaphoreType.DMA((3,)),
            pltpu.SemaphoreType.DMA((3,)),
        ],
    )
    return f(y, er, zeros)


def _tc_prep_body(x_ref, w_ref, deg_ref, y_ref):
    xw = jnp.dot(x_ref[...], w_ref[...], preferred_element_type=jnp.float32)
    deg = deg_ref[0, :] + deg_ref[1, :] + 1.0
    dinv = lax.rsqrt(deg)
    y_ref[...] = xw * dinv[:, None]


def _tc_prep(x, W, deg_part):
    return pl.pallas_call(
        _tc_prep_body,
        grid=(GRID,),
        in_specs=[
            pl.BlockSpec((BLK, D), lambda i: (i, 0)),
            pl.BlockSpec((D, D), lambda i: (0, 0)),
            pl.BlockSpec((NC, BLK), lambda i: (0, i)),
        ],
        out_specs=pl.BlockSpec((BLK, D), lambda i: (i, 0)),
        out_shape=jax.ShapeDtypeStruct((N, D), jnp.float32),
    )(x, W, deg_part)


def _p1(i):
    return jnp.minimum(i, GRID - 1)


def _p2(i):
    return jnp.maximum(i - GRID, 0)
def _tc_fin_body(acc_ref, y_ref, deg_ref, b_ref, x_ref, w_ref, bias_ref,
                 ms_ref, o_ref, h_sc, s1, s2):
    i = pl.program_id(0)
    ip = lax.rem(i, GRID)

    @pl.when(i == 0)
    def _():
        s1[...] = jnp.zeros_like(s1)
        s2[...] = jnp.zeros_like(s2)

    @pl.when(i < GRID)
    def _():
        deg = deg_ref[0, :] + deg_ref[1, :] + 1.0
        dinv = lax.rsqrt(deg)
        h = dinv[:, None] * (acc_ref[0] + acc_ref[1] + y_ref[...]) + b_ref[...]
        row = ip * BLK + lax.broadcasted_iota(jnp.int32, (BLK, D), 0)
        hm = jnp.where(row < N, h, 0.0)
        h_sc[ip] = h
        s1[...] += jnp.sum(hm, axis=0, keepdims=True)
        s2[...] += jnp.sum(hm * hm, axis=0, keepdims=True)

    @pl.when(i >= GRID)
    def _():
        m = s1[0, :] * (1.0 / N)
        q = s2[0, :] * (1.0 / N)
        ms = ms_ref[0, :]
        mm = m * ms
        var = q - 2.0 * mm * m + mm * mm
        scale = w_ref[0, :] * lax.rsqrt(var + EPS)
        h = h_sc[ip]
        o = (h - mm[None, :]) * scale[None, :] + bias_ref[0, :][None, :]
        o_ref[...] = jnp.maximum(o, 0.0) + x_ref[...]


def _tc_finish(acc_part, y, deg_part, b2, x, gw2, gb2, gms2):
    return pl.pallas_call(
        _tc_fin_body,
        grid=(2 * GRID,),
        in_specs=[
            pl.BlockSpec((NC, BLK, D), lambda i: (0, _p1(i), 0)),
            pl.BlockSpec((BLK, D), lambda i: (_p1(i), 0)),
            pl.BlockSpec((NC, BLK), lambda i: (0, _p1(i))),
            pl.BlockSpec((1, D), lambda i: (0, 0)),
            pl.BlockSpec((BLK, D), lambda i: (_p2(i), 0)),
            pl.BlockSpec((1, D), lambda i: (0, 0)),
            pl.BlockSpec((1, D), lambda i: (0, 0)),
            pl.BlockSpec((1, D), lambda i: (0, 0)),
        ],
        out_specs=pl.BlockSpec((BLK, D), lambda i: (_p2(i), 0)),
        out_shape=jax.ShapeDtypeStruct((N, D), jnp.float32),
        scratch_shapes=[
            pltpu.VMEM((GRID, BLK, D), jnp.float32),
            pltpu.VMEM((1, D), jnp.float32),
            pltpu.VMEM((1, D), jnp.float32),
        ],
    )(acc_part, y, deg_part, b2, x, gw2, gb2, gms2)


def kernel(x, edge_index, W, b, gn_weight, gn_bias, gn_mean_scale):
    er = edge_index.reshape(2, NW, NCHUNK, CHUNK)

    ones = jnp.ones((CHUNK,), jnp.float32)
    zcol = jnp.zeros((ROWS_PER_TILE,), jnp.float32)
    zeros = jnp.zeros((ROWS_PER_TILE, D), jnp.float32)

    deg_part = _sc_deg(er, ones, zcol)
    y = _tc_prep(x, W, deg_part)
    acc_part = _sc_scatter(y, er, zeros)
    return _tc_finish(acc_part, y, deg_part, b.reshape(1, D), x,
                      gn_weight.reshape(1, D), gn_bias.reshape(1, D),
                      gn_mean_scale.reshape(1, D))


# speedup vs baseline: 1.2798x; 1.0484x over previous
import functools

import jax
import jax.numpy as jnp
from jax import lax
from jax.experimental import pallas as pl
from jax.experimental.pallas import tpu as pltpu
from jax.experimental.pallas import tpu_sc as plsc

N = 10000
E = 320000
D = 128
EPS = 1e-5

NC = 2
NS = 16
NW = NC * NS
CHUNK = 100
E_PER_TILE = E // NW
NCHUNK = -(-E_PER_TILE // CHUNK)
EDGE_PAD = NCHUNK * CHUNK - E_PER_TILE
ROWS_PER_TILE = 640
N_PAD = 10240

BLK = 2048
GRID = N_PAD // BLK


def _sc_deg_body(er_hbm, ones_hbm, zcol_hbm, deg_out, dst_v, ones_v, deg_sh,
                 ssem):
    c = lax.axis_index("c")
    s = lax.axis_index("s")
    wid = c * NS + s
    pltpu.sync_copy(er_hbm.at[1, wid], dst_v)
    pltpu.sync_copy(ones_hbm, ones_v)
    pltpu.sync_copy(zcol_hbm, deg_sh.at[pl.ds(s * ROWS_PER_TILE, ROWS_PER_TILE)])
    plsc.subcore_barrier()

    pltpu.async_copy(ones_v, deg_sh.at[dst_v.at[0]], ssem.at[0], add=True)

    def step(j, carry):
        @pl.when(j + 1 < NCHUNK)
        def _():
            pltpu.async_copy(ones_v, deg_sh.at[dst_v.at[j + 1]],
                             ssem.at[_even(j + 1)], add=True)

        pltpu.make_async_copy(ones_v, deg_sh.at[dst_v.at[j]],
                              ssem.at[_even(j)]).wait()
        return carry

    lax.fori_loop(0, NCHUNK, step, 0)
    plsc.subcore_barrier()
    sl = pl.ds(s * ROWS_PER_TILE, ROWS_PER_TILE)
    pltpu.sync_copy(deg_sh.at[sl], deg_out.at[c, sl])


def _even(j):
    return lax.rem(j, 2)


def _mod3(j):
    return lax.rem(j, 3)


def _sc_deg(er, ones, zcol):
    mesh = plsc.VectorSubcoreMesh(
        core_axis_name="c", subcore_axis_name="s", num_cores=NC, num_subcores=NS
    )
    f = pl.kernel(
        _sc_deg_body,
        out_type=jax.ShapeDtypeStruct((NC, N_PAD), jnp.float32),
        mesh=mesh,
        scratch_types=[
            pltpu.VMEM((NCHUNK, CHUNK), jnp.int32),
            pltpu.VMEM((CHUNK,), jnp.float32),
            pltpu.VMEM_SHARED((N_PAD,), jnp.float32),
            pltpu.SemaphoreType.DMA((2,)),
        ],
    )
    return f(er, ones, zcol)


def _mod4(j):
    return lax.rem(j, 4)


def _sc_scat_body(y_hbm, er_hbm, zeros_hbm, out_hbm,
                  srcbuf, dstbuf, rows_v, acc_sh, srcsem, dstsem, gsem, ssem):
    c = lax.axis_index("c")
    s = lax.axis_index("s")
    wid = c * NS + s

    def fetch(j, islot):
        pltpu.async_copy(er_hbm.at[0, wid, j], srcbuf.at[islot],
                         srcsem.at[islot])
        pltpu.async_copy(er_hbm.at[1, wid, j], dstbuf.at[islot],
                         dstsem.at[islot])

    def gather(j, islot, rslot):
        pltpu.make_async_copy(er_hbm.at[0, wid, j], srcbuf.at[islot],
                              srcsem.at[islot]).wait()
        pltpu.async_copy(y_hbm.at[srcbuf.at[islot]], rows_v.at[rslot],
                         gsem.at[rslot])

    pltpu.async_copy(zeros_hbm,
                     acc_sh.at[pl.ds(s * ROWS_PER_TILE, ROWS_PER_TILE)],
                     ssem.at[2])
    fetch(0, 0)
    fetch(1, 1)
    gather(0, 0, 0)
    pltpu.make_async_copy(zeros_hbm,
                          acc_sh.at[pl.ds(s * ROWS_PER_TILE, ROWS_PER_TILE)],
                          ssem.at[2]).wait()
    plsc.subcore_barrier()

    def step(j, carry):
        @pl.when(j >= 2)
        def _():
            pslot = _mod3(j - 2)
            pltpu.make_async_copy(rows_v.at[pslot],
                                  acc_sh.at[dstbuf.at[_mod4(j - 2)]],
                                  ssem.at[pslot]).wait()

        @pl.when(j + 2 < NCHUNK)
        def _():
            fetch(j + 2, _mod4(j + 2))

        @pl.when(j + 1 < NCHUNK)
        def _():
            gather(j + 1, _mod4(j + 1), _mod3(j + 1))

        rslot = _mod3(j)
        islot = _mod4(j)
        pltpu.make_async_copy(y_hbm.at[srcbuf.at[islot]], rows_v.at[rslot],
                              gsem.at[rslot]).wait()
        pltpu.make_async_copy(er_hbm.at[1, wid, j], dstbuf.at[islot],
                              dstsem.at[islot]).wait()
        pltpu.async_copy(rows_v.at[rslot], acc_sh.at[dstbuf.at[islot]],
                         ssem.at[rslot], add=True)
        return carry

    lax.fori_loop(0, NCHUNK, step, 0)
    for j in (NCHUNK - 2, NCHUNK - 1):
        pltpu.make_async_copy(rows_v.at[j % 3],
                              acc_sh.at[dstbuf.at[j % 4]],
                              ssem.at[j % 3]).wait()
    plsc.subcore_barrier()
    sl = pl.ds(s * ROWS_PER_TILE, ROWS_PER_TILE)
    pltpu.sync_copy(acc_sh.at[sl], out_hbm.at[c, sl])


def _sc_scatter(y, er, zeros):
    mesh = plsc.VectorSubcoreMesh(
        core_axis_name="c", subcore_axis_name="s", num_cores=NC, num_subcores=NS
    )
    f = pl.kernel(
        _sc_scat_body,
        out_type=jax.ShapeDtypeStruct((NC, N_PAD, D), jnp.float32),
        mesh=mesh,
        scratch_types=[
            pltpu.VMEM((4, CHUNK), jnp.int32),
            pltpu.VMEM((4, CHUNK), jnp.int32),
            pltpu.VMEM((3, CHUNK, D), jnp.float32),
            pltpu.VMEM_SHARED((N_PAD, D), jnp.float32),
            pltpu.SemaphoreType.DMA((4,)),
            pltpu.SemaphoreType.DMA((4,)),
            pltpu.SemaphoreType.DMA((3,)),
            pltpu.SemaphoreType.DMA((3,)),
        ],
    )
    return f(y, er, zeros)


def _tc_prep_body(x_ref, w_ref, deg_ref, y_ref):
    xw = jnp.dot(x_ref[...], w_ref[...], preferred_element_type=jnp.float32)
    deg = deg_ref[0, :] + deg_ref[1, :] + 1.0
    dinv = lax.rsqrt(deg)
    y_ref[...] = xw * dinv[:, None]


def _tc_prep(x, W, deg_part):
    return pl.pallas_call(
        _tc_prep_body,
        grid=(GRID,),
        in_specs=[
            pl.BlockSpec((BLK, D), lambda i: (i, 0)),
            pl.BlockSpec((D, D), lambda i: (0, 0)),
            pl.BlockSpec((NC, BLK), lambda i: (0, i)),
        ],
        out_specs=pl.BlockSpec((BLK, D), lambda i: (i, 0)),
        out_shape=jax.ShapeDtypeStruct((N, D), jnp.float32),
    )(x, W, deg_part)


def _p1(i):
    return jnp.minimum(i, GRID - 1)


def _p2(i):
    return jnp.maximum(i - GRID, 0)
def _tc_fin_body(acc_ref, y_ref, deg_ref, b_ref, x_ref, w_ref, bias_ref,
                 ms_ref, o_ref, h_sc, s1, s2):
    i = pl.program_id(0)
    ip = lax.rem(i, GRID)

    @pl.when(i == 0)
    def _():
        s1[...] = jnp.zeros_like(s1)
        s2[...] = jnp.zeros_like(s2)

    @pl.when(i < GRID)
    def _():
        deg = deg_ref[0, :] + deg_ref[1, :] + 1.0
        dinv = lax.rsqrt(deg)
        h = dinv[:, None] * (acc_ref[0] + acc_ref[1] + y_ref[...]) + b_ref[...]
        row = ip * BLK + lax.broadcasted_iota(jnp.int32, (BLK, D), 0)
        hm = jnp.where(row < N, h, 0.0)
        h_sc[ip] = h
        s1[...] += jnp.sum(hm, axis=0, keepdims=True)
        s2[...] += jnp.sum(hm * hm, axis=0, keepdims=True)

    @pl.when(i >= GRID)
    def _():
        m = s1[0, :] * (1.0 / N)
        q = s2[0, :] * (1.0 / N)
        ms = ms_ref[0, :]
        mm = m * ms
        var = q - 2.0 * mm * m + mm * mm
        scale = w_ref[0, :] * lax.rsqrt(var + EPS)
        h = h_sc[ip]
        o = (h - mm[None, :]) * scale[None, :] + bias_ref[0, :][None, :]
        o_ref[...] = jnp.maximum(o, 0.0) + x_ref[...]


def _tc_finish(acc_part, y, deg_part, b2, x, gw2, gb2, gms2):
    return pl.pallas_call(
        _tc_fin_body,
        grid=(2 * GRID,),
        in_specs=[
            pl.BlockSpec((NC, BLK, D), lambda i: (0, _p1(i), 0)),
            pl.BlockSpec((BLK, D), lambda i: (_p1(i), 0)),
            pl.BlockSpec((NC, BLK), lambda i: (0, _p1(i))),
            pl.BlockSpec((1, D), lambda i: (0, 0)),
            pl.BlockSpec((BLK, D), lambda i: (_p2(i), 0)),
            pl.BlockSpec((1, D), lambda i: (0, 0)),
            pl.BlockSpec((1, D), lambda i: (0, 0)),
            pl.BlockSpec((1, D), lambda i: (0, 0)),
        ],
        out_specs=pl.BlockSpec((BLK, D), lambda i: (_p2(i), 0)),
        out_shape=jax.ShapeDtypeStruct((N, D), jnp.float32),
        scratch_shapes=[
            pltpu.VMEM((GRID, BLK, D), jnp.float32),
            pltpu.VMEM((1, D), jnp.float32),
            pltpu.VMEM((1, D), jnp.float32),
        ],
    )(acc_part, y, deg_part, b2, x, gw2, gb2, gms2)


def kernel(x, edge_index, W, b, gn_weight, gn_bias, gn_mean_scale):
    er = edge_index.reshape(2, NW, NCHUNK, CHUNK)

    ones = jnp.ones((CHUNK,), jnp.float32)
    zcol = jnp.zeros((ROWS_PER_TILE,), jnp.float32)
    zeros = jnp.zeros((ROWS_PER_TILE, D), jnp.float32)

    deg_part = _sc_deg(er, ones, zcol)
    y = _tc_prep(x, W, deg_part)
    acc_part = _sc_scatter(y, er, zeros)
    return _tc_finish(acc_part, y, deg_part, b.reshape(1, D), x,
                      gn_weight.reshape(1, D), gn_bias.reshape(1, D),
                      gn_mean_scale.reshape(1, D))


# speedup vs baseline: 1.3125x; 1.0255x over previous
import functools

import jax
import jax.numpy as jnp
from jax import lax
from jax.experimental import pallas as pl
from jax.experimental.pallas import tpu as pltpu
from jax.experimental.pallas import tpu_sc as plsc

N = 10000
E = 320000
D = 128
EPS = 1e-5

NC = 2
NS = 16
NW = NC * NS
CHUNK = 100
E_PER_TILE = E // NW
NCHUNK = -(-E_PER_TILE // CHUNK)
EDGE_PAD = NCHUNK * CHUNK - E_PER_TILE
ROWS_PER_TILE = 640
N_PAD = 10240

BLK = 5120
GRID = N_PAD // BLK


def _sc_deg_body(er_hbm, ones_hbm, zcol_hbm, deg_out, dst_v, ones_v, deg_sh,
                 ssem):
    c = lax.axis_index("c")
    s = lax.axis_index("s")
    wid = c * NS + s
    pltpu.sync_copy(er_hbm.at[1, wid], dst_v)
    pltpu.sync_copy(ones_hbm, ones_v)
    pltpu.sync_copy(zcol_hbm, deg_sh.at[pl.ds(s * ROWS_PER_TILE, ROWS_PER_TILE)])
    plsc.subcore_barrier()

    pltpu.async_copy(ones_v, deg_sh.at[dst_v.at[0]], ssem.at[0], add=True)

    def step(j, carry):
        @pl.when(j + 1 < NCHUNK)
        def _():
            pltpu.async_copy(ones_v, deg_sh.at[dst_v.at[j + 1]],
                             ssem.at[_even(j + 1)], add=True)

        pltpu.make_async_copy(ones_v, deg_sh.at[dst_v.at[j]],
                              ssem.at[_even(j)]).wait()
        return carry

    lax.fori_loop(0, NCHUNK, step, 0)
    plsc.subcore_barrier()
    sl = pl.ds(s * ROWS_PER_TILE, ROWS_PER_TILE)
    pltpu.sync_copy(deg_sh.at[sl], deg_out.at[c, sl])


def _even(j):
    return lax.rem(j, 2)


def _mod3(j):
    return lax.rem(j, 3)


def _sc_deg(er, ones, zcol):
    mesh = plsc.VectorSubcoreMesh(
        core_axis_name="c", subcore_axis_name="s", num_cores=NC, num_subcores=NS
    )
    f = pl.kernel(
        _sc_deg_body,
        out_type=jax.ShapeDtypeStruct((NC, N_PAD), jnp.float32),
        mesh=mesh,
        scratch_types=[
            pltpu.VMEM((NCHUNK, CHUNK), jnp.int32),
            pltpu.VMEM((CHUNK,), jnp.float32),
            pltpu.VMEM_SHARED((N_PAD,), jnp.float32),
            pltpu.SemaphoreType.DMA((2,)),
        ],
    )
    return f(er, ones, zcol)


def _mod4(j):
    return lax.rem(j, 4)


def _sc_scat_body(y_hbm, er_hbm, zeros_hbm, out_hbm,
                  srcbuf, dstbuf, rows_v, acc_sh, srcsem, dstsem, gsem, ssem):
    c = lax.axis_index("c")
    s = lax.axis_index("s")
    wid = c * NS + s

    def fetch(j, islot):
        pltpu.async_copy(er_hbm.at[0, wid, j], srcbuf.at[islot],
                         srcsem.at[islot])
        pltpu.async_copy(er_hbm.at[1, wid, j], dstbuf.at[islot],
                         dstsem.at[islot])

    def gather(j, islot, rslot):
        pltpu.make_async_copy(er_hbm.at[0, wid, j], srcbuf.at[islot],
                              srcsem.at[islot]).wait()
        pltpu.async_copy(y_hbm.at[srcbuf.at[islot]], rows_v.at[rslot],
                         gsem.at[rslot])

    pltpu.async_copy(zeros_hbm,
                     acc_sh.at[pl.ds(s * ROWS_PER_TILE, ROWS_PER_TILE)],
                     ssem.at[2])
    fetch(0, 0)
    fetch(1, 1)
    gather(0, 0, 0)
    pltpu.make_async_copy(zeros_hbm,
                          acc_sh.at[pl.ds(s * ROWS_PER_TILE, ROWS_PER_TILE)],
                          ssem.at[2]).wait()
    plsc.subcore_barrier()

    def step(j, carry):
        @pl.when(j >= 2)
        def _():
            pslot = _mod3(j - 2)
            pltpu.make_async_copy(rows_v.at[pslot],
                                  acc_sh.at[dstbuf.at[_mod4(j - 2)]],
                                  ssem.at[pslot]).wait()

        @pl.when(j + 2 < NCHUNK)
        def _():
            fetch(j + 2, _mod4(j + 2))

        @pl.when(j + 1 < NCHUNK)
        def _():
            gather(j + 1, _mod4(j + 1), _mod3(j + 1))

        rslot = _mod3(j)
        islot = _mod4(j)
        pltpu.make_async_copy(y_hbm.at[srcbuf.at[islot]], rows_v.at[rslot],
                              gsem.at[rslot]).wait()
        pltpu.make_async_copy(er_hbm.at[1, wid, j], dstbuf.at[islot],
                              dstsem.at[islot]).wait()
        pltpu.async_copy(rows_v.at[rslot], acc_sh.at[dstbuf.at[islot]],
                         ssem.at[rslot], add=True)
        return carry

    lax.fori_loop(0, NCHUNK, step, 0)
    for j in (NCHUNK - 2, NCHUNK - 1):
        pltpu.make_async_copy(rows_v.at[j % 3],
                              acc_sh.at[dstbuf.at[j % 4]],
                              ssem.at[j % 3]).wait()
    plsc.subcore_barrier()
    sl = pl.ds(s * ROWS_PER_TILE, ROWS_PER_TILE)
    pltpu.sync_copy(acc_sh.at[sl], out_hbm.at[c, sl])


def _sc_scatter(y, er, zeros):
    mesh = plsc.VectorSubcoreMesh(
        core_axis_name="c", subcore_axis_name="s", num_cores=NC, num_subcores=NS
    )
    f = pl.kernel(
        _sc_scat_body,
        out_type=jax.ShapeDtypeStruct((NC, N_PAD, D), jnp.float32),
        mesh=mesh,
        scratch_types=[
            pltpu.VMEM((4, CHUNK), jnp.int32),
            pltpu.VMEM((4, CHUNK), jnp.int32),
            pltpu.VMEM((3, CHUNK, D), jnp.float32),
            pltpu.VMEM_SHARED((N_PAD, D), jnp.float32),
            pltpu.SemaphoreType.DMA((4,)),
            pltpu.SemaphoreType.DMA((4,)),
            pltpu.SemaphoreType.DMA((3,)),
            pltpu.SemaphoreType.DMA((3,)),
        ],
    )
    return f(y, er, zeros)


def _tc_prep_body(x_ref, w_ref, deg_ref, y_ref):
    xw = jnp.dot(x_ref[...], w_ref[...], preferred_element_type=jnp.float32)
    deg = deg_ref[0, :] + deg_ref[1, :] + 1.0
    dinv = lax.rsqrt(deg)
    y_ref[...] = xw * dinv[:, None]


def _tc_prep(x, W, deg_part):
    return pl.pallas_call(
        _tc_prep_body,
        grid=(GRID,),
        in_specs=[
            pl.BlockSpec((BLK, D), lambda i: (i, 0)),
            pl.BlockSpec((D, D), lambda i: (0, 0)),
            pl.BlockSpec((NC, BLK), lambda i: (0, i)),
        ],
        out_specs=pl.BlockSpec((BLK, D), lambda i: (i, 0)),
        out_shape=jax.ShapeDtypeStruct((N, D), jnp.float32),
    )(x, W, deg_part)


def _p1(i):
    return jnp.minimum(i, GRID - 1)


def _p2(i):
    return jnp.maximum(i - GRID, 0)
def _tc_fin_body(acc_ref, y_ref, deg_ref, b_ref, x_ref, w_ref, bias_ref,
                 ms_ref, o_ref, h_sc, s1, s2):
    i = pl.program_id(0)
    ip = lax.rem(i, GRID)

    @pl.when(i == 0)
    def _():
        s1[...] = jnp.zeros_like(s1)
        s2[...] = jnp.zeros_like(s2)

    @pl.when(i < GRID)
    def _():
        deg = deg_ref[0, :] + deg_ref[1, :] + 1.0
        dinv = lax.rsqrt(deg)
        h = dinv[:, None] * (acc_ref[0] + acc_ref[1] + y_ref[...]) + b_ref[...]
        row = ip * BLK + lax.broadcasted_iota(jnp.int32, (BLK, D), 0)
        hm = jnp.where(row < N, h, 0.0)
        h_sc[ip] = h
        s1[...] += jnp.sum(hm, axis=0, keepdims=True)
        s2[...] += jnp.sum(hm * hm, axis=0, keepdims=True)

    @pl.when(i >= GRID)
    def _():
        m = s1[0, :] * (1.0 / N)
        q = s2[0, :] * (1.0 / N)
        ms = ms_ref[0, :]
        mm = m * ms
        var = q - 2.0 * mm * m + mm * mm
        scale = w_ref[0, :] * lax.rsqrt(var + EPS)
        h = h_sc[ip]
        o = (h - mm[None, :]) * scale[None, :] + bias_ref[0, :][None, :]
        o_ref[...] = jnp.maximum(o, 0.0) + x_ref[...]


def _tc_finish(acc_part, y, deg_part, b2, x, gw2, gb2, gms2):
    return pl.pallas_call(
        _tc_fin_body,
        grid=(2 * GRID,),
        in_specs=[
            pl.BlockSpec((NC, BLK, D), lambda i: (0, _p1(i), 0)),
            pl.BlockSpec((BLK, D), lambda i: (_p1(i), 0)),
            pl.BlockSpec((NC, BLK), lambda i: (0, _p1(i))),
            pl.BlockSpec((1, D), lambda i: (0, 0)),
            pl.BlockSpec((BLK, D), lambda i: (_p2(i), 0)),
            pl.BlockSpec((1, D), lambda i: (0, 0)),
            pl.BlockSpec((1, D), lambda i: (0, 0)),
            pl.BlockSpec((1, D), lambda i: (0, 0)),
        ],
        out_specs=pl.BlockSpec((BLK, D), lambda i: (_p2(i), 0)),
        out_shape=jax.ShapeDtypeStruct((N, D), jnp.float32),
        scratch_shapes=[
            pltpu.VMEM((GRID, BLK, D), jnp.float32),
            pltpu.VMEM((1, D), jnp.float32),
            pltpu.VMEM((1, D), jnp.float32),
        ],
    )(acc_part, y, deg_part, b2, x, gw2, gb2, gms2)


def kernel(x, edge_index, W, b, gn_weight, gn_bias, gn_mean_scale):
    er = edge_index.reshape(2, NW, NCHUNK, CHUNK)

    ones = jnp.ones((CHUNK,), jnp.float32)
    zcol = jnp.zeros((ROWS_PER_TILE,), jnp.float32)
    zeros = jnp.zeros((ROWS_PER_TILE, D), jnp.float32)

    deg_part = _sc_deg(er, ones, zcol)
    y = _tc_prep(x, W, deg_part)
    acc_part = _sc_scatter(y, er, zeros)
    return _tc_finish(acc_part, y, deg_part, b.reshape(1, D), x,
                      gn_weight.reshape(1, D), gn_bias.reshape(1, D),
                      gn_mean_scale.reshape(1, D))
